# baseline (speedup 1.0000x reference)
import functools

import jax
import jax.numpy as jnp
from jax import lax
from jax.experimental import pallas as pl
from jax.experimental.pallas import tpu as pltpu
from jax.experimental.pallas import tpu_sc as plsc

USER_NUM = 1000000
ITEM_NUM = 1000000
D = 64
B = 16384
NC, NS, L = 2, 16, 16
NW = NC * NS
BPW = B // NW
CHUNK = 128
NCHUNK = BPW // CHUNK


def _body(user_hbm, item_i_hbm, item_j_hbm, eu_hbm, ei_hbm,
          out_i_hbm, out_j_hbm,
          uidx_v, iidx_v, jidx_v, eu_v, ei_v, ej_v, pi_v, pj_v, sem):
    wid = lax.axis_index("s") * NC + lax.axis_index("c")
    base = wid * BPW

    pltpu.sync_copy(user_hbm.at[pl.ds(base, BPW)], uidx_v)
    pltpu.sync_copy(item_i_hbm.at[pl.ds(base, BPW)], iidx_v)
    pltpu.sync_copy(item_j_hbm.at[pl.ds(base, BPW)], jidx_v)

    eu_2d, ei_2d, ej_2d = eu_v, ei_v, ej_v
    copies = []
    for c in range(NCHUNK):
        sl = pl.ds(c * CHUNK, CHUNK)
        copies.append(pltpu.async_copy(eu_hbm.at[uidx_v.at[sl]], eu_2d.at[sl], sem))
        copies.append(pltpu.async_copy(ei_hbm.at[iidx_v.at[sl]], ei_2d.at[sl], sem))
        copies.append(pltpu.async_copy(ei_hbm.at[jidx_v.at[sl]], ej_2d.at[sl], sem))
    for cp in copies:
        cp.wait()

    row0 = lax.iota(jnp.int32, 16)

    def blk_body(blk, carry):
        rows = blk * 16 + row0
        acc_i = jnp.zeros((16,), jnp.float32)
        acc_j = jnp.zeros((16,), jnp.float32)
        for f in range(D):
            cols = jnp.full((16,), f, jnp.int32)
            u = plsc.load_gather(eu_v, [rows, cols])
            a = plsc.load_gather(ei_v, [rows, cols])
            b = plsc.load_gather(ej_v, [rows, cols])
            acc_i = acc_i + u * a
            acc_j = acc_j + u * b
        pi_v[pl.ds(blk * 16, 16)] = acc_i
        pj_v[pl.ds(blk * 16, 16)] = acc_j
        return carry

    lax.fori_loop(0, BPW // 16, blk_body, 0)

    pltpu.sync_copy(pi_v, out_i_hbm.at[pl.ds(base, BPW)])
    pltpu.sync_copy(pj_v, out_j_hbm.at[pl.ds(base, BPW)])


@jax.jit
def _run(user, item_i, item_j, embed_user, embed_item):
    mesh = plsc.VectorSubcoreMesh(core_axis_name="c", subcore_axis_name="s")
    k = pl.kernel(
        _body,
        out_type=(
            jax.ShapeDtypeStruct((B,), jnp.float32),
            jax.ShapeDtypeStruct((B,), jnp.float32),
        ),
        mesh=mesh,
        scratch_types=[
            pltpu.VMEM((BPW,), jnp.int32),
            pltpu.VMEM((BPW,), jnp.int32),
            pltpu.VMEM((BPW,), jnp.int32),
            pltpu.VMEM((BPW, D), jnp.float32),
            pltpu.VMEM((BPW, D), jnp.float32),
            pltpu.VMEM((BPW, D), jnp.float32),
            pltpu.VMEM((BPW,), jnp.float32),
            pltpu.VMEM((BPW,), jnp.float32),
            pltpu.SemaphoreType.DMA,
        ],
        compiler_params=pltpu.CompilerParams(
            needs_layout_passes=False, use_tc_tiling_on_sc=False),
    )
    return k(user, item_i, item_j, embed_user, embed_item)


def kernel(user, item_i, item_j, embed_user, embed_item):
    return _run(user.astype(jnp.int32), item_i.astype(jnp.int32),
                item_j.astype(jnp.int32), embed_user, embed_item)


# speedup vs baseline: 1.5123x; 1.5123x over previous
import functools

import jax
import jax.numpy as jnp
from jax import lax
from jax.experimental import pallas as pl
from jax.experimental.pallas import tpu as pltpu
from jax.experimental.pallas import tpu_sc as plsc

D = 64
B = 16384
NC, NS, L = 2, 16, 16
NW = NC * NS
BPW = B // NW
NBLK = BPW // L


def _body(user_hbm, item_i_hbm, item_j_hbm, eu_hbm, ei_hbm,
          out_i_hbm, out_j_hbm,
          uidx_v, iidx_v, jidx_v, eu_r, ei_r, ej_r, pi_v, pj_v, sem):
    wid = lax.axis_index("s") * NC + lax.axis_index("c")
    base = wid * BPW

    pltpu.sync_copy(user_hbm.at[pl.ds(base, BPW)], uidx_v)
    pltpu.sync_copy(item_i_hbm.at[pl.ds(base, BPW)], iidx_v)
    pltpu.sync_copy(item_j_hbm.at[pl.ds(base, BPW)], jidx_v)

    lane = lax.iota(jnp.int32, L)

    def blk_body(blk, carry):
        sl = pl.ds(blk * L, L)
        uu = uidx_v[sl]
        iiv = iidx_v[sl]
        jjv = jidx_v[sl]
        handles = []
        for k in range(L):
            iu = uu[k]
            ii = iiv[k]
            ij = jjv[k]
            dk = pl.ds(k, 1)
            handles.append(pltpu.async_copy(eu_hbm.at[pl.ds(iu, 1)], eu_r.at[dk], sem))
            handles.append(pltpu.async_copy(ei_hbm.at[pl.ds(ii, 1)], ei_r.at[dk], sem))
            handles.append(pltpu.async_copy(ei_hbm.at[pl.ds(ij, 1)], ej_r.at[dk], sem))
        for h in handles:
            h.wait()
        acc_i = jnp.zeros((L,), jnp.float32)
        acc_j = jnp.zeros((L,), jnp.float32)
        for f in range(D):
            cols = jnp.full((L,), f, jnp.int32)
            u = plsc.load_gather(eu_r, [lane, cols])
            a = plsc.load_gather(ei_r, [lane, cols])
            b = plsc.load_gather(ej_r, [lane, cols])
            acc_i = acc_i + u * a
            acc_j = acc_j + u * b
        pi_v[sl] = acc_i
        pj_v[sl] = acc_j
        return carry

    lax.fori_loop(0, NBLK, blk_body, 0)

    pltpu.sync_copy(pi_v, out_i_hbm.at[pl.ds(base, BPW)])
    pltpu.sync_copy(pj_v, out_j_hbm.at[pl.ds(base, BPW)])


@jax.jit
def _run(user, item_i, item_j, embed_user, embed_item):
    mesh = plsc.VectorSubcoreMesh(core_axis_name="c", subcore_axis_name="s")
    k = pl.kernel(
        _body,
        out_type=(
            jax.ShapeDtypeStruct((B,), jnp.float32),
            jax.ShapeDtypeStruct((B,), jnp.float32),
        ),
        mesh=mesh,
        scratch_types=[
            pltpu.VMEM((BPW,), jnp.int32),
            pltpu.VMEM((BPW,), jnp.int32),
            pltpu.VMEM((BPW,), jnp.int32),
            pltpu.VMEM((L, D), jnp.float32),
            pltpu.VMEM((L, D), jnp.float32),
            pltpu.VMEM((L, D), jnp.float32),
            pltpu.VMEM((BPW,), jnp.float32),
            pltpu.VMEM((BPW,), jnp.float32),
            pltpu.SemaphoreType.DMA,
        ],
        compiler_params=pltpu.CompilerParams(needs_layout_passes=False),
    )
    return k(user, item_i, item_j, embed_user, embed_item)


def kernel(user, item_i, item_j, embed_user, embed_item):
    return _run(user.astype(jnp.int32), item_i.astype(jnp.int32),
                item_j.astype(jnp.int32), embed_user, embed_item)
